# Initial kernel scaffold; baseline (speedup 1.0000x reference)
import jax
import jax.numpy as jnp
from jax.experimental import pallas as pl


def kernel(depth_map, x_ray):
    raise NotImplementedError("write your pallas kernel here")


# speedup vs baseline: 26.0818x; 26.0818x over previous
import jax
import jax.numpy as jnp
import numpy as np
from jax.experimental import pallas as pl

_D = 128
_DB = 32

_G1 = float(np.exp(-0.5))
_G2 = float(np.exp(-2.0))
_K0 = 1.0 + 2.0 * _G1
_K1 = 1.0 + _G1 + _G2
_K2 = _G1 + _G2
_K3 = _G2


def _splat_pool_kernel(depth_ref, xray_ref, out_ref):
    db = pl.program_id(1)
    depth = depth_ref[0, 0]
    inten = xray_ref[0, 0]
    H, W = depth.shape
    di = jnp.clip((depth * (_D - 1)).astype(jnp.int32), 0, _D - 1)

    d0 = db * _DB + jax.lax.broadcasted_iota(jnp.int32, (_DB, 1, 1), 0)
    a = jnp.abs(d0 - di[None, :, :])
    w = jnp.where(a == 0, _K0,
        jnp.where(a == 1, _K1,
        jnp.where(a == 2, _K2,
        jnp.where(a == 3, _K3, 0.0)))).astype(jnp.float32)

    edge_lo = (d0 == 0).astype(jnp.float32)
    edge_hi = (d0 == _D - 1).astype(jnp.float32)
    g_lo = jnp.where(di == 0, _G1, jnp.where(di == 1, _G2, 0.0))
    g_hi = jnp.where(di == _D - 1, _G1, jnp.where(di == _D - 2, _G2, 0.0))
    w = w - edge_lo * g_lo[None] - edge_hi * g_hi[None]

    p = w * inten[None, :, :]

    z_h = jnp.zeros((_DB, 1, W), jnp.float32)
    t = p + jnp.concatenate([z_h, p[:, :-1, :]], axis=1) \
          + jnp.concatenate([p[:, 1:, :], z_h], axis=1)
    z_w = jnp.zeros((_DB, H, 1), jnp.float32)
    s = t + jnp.concatenate([z_w, t[:, :, :-1]], axis=2) \
          + jnp.concatenate([t[:, :, 1:], z_w], axis=2)
    out_ref[0] = s * (1.0 / 27.0)


def kernel(depth_map, x_ray):
    B, _, H, W = depth_map.shape
    out = pl.pallas_call(
        _splat_pool_kernel,
        grid=(B, _D // _DB),
        in_specs=[
            pl.BlockSpec((1, 1, H, W), lambda b, d: (b, 0, 0, 0)),
            pl.BlockSpec((1, 1, H, W), lambda b, d: (b, 0, 0, 0)),
        ],
        out_specs=pl.BlockSpec((1, _DB, H, W), lambda b, d: (b, d, 0, 0)),
        out_shape=jax.ShapeDtypeStruct((B, _D, H, W), jnp.float32),
    )(depth_map, x_ray)
    return out[:, None]


# speedup vs baseline: 29.3528x; 1.1254x over previous
import jax
import jax.numpy as jnp
import numpy as np
from jax.experimental import pallas as pl

_D = 128
_DB = 32

_G1 = float(np.exp(-0.5))
_G2 = float(np.exp(-2.0))
_K0 = (1.0 + 2.0 * _G1) / 27.0
_K1 = (1.0 + _G1 + _G2) / 27.0
_K2 = (_G1 + _G2) / 27.0
_K3 = _G2 / 27.0


def _box3x3(p, zh, zw):
    t = p + jnp.concatenate([zh, p[:, :-1, :]], axis=1) \
          + jnp.concatenate([p[:, 1:, :], zh], axis=1)
    return t + jnp.concatenate([zw, t[:, :, :-1]], axis=2) \
             + jnp.concatenate([t[:, :, 1:], zw], axis=2)


def _splat_pool_kernel(depth_ref, xray_ref, out_ref):
    db = pl.program_id(1)
    nd = pl.num_programs(1)
    depth = depth_ref[0, 0]
    inten = xray_ref[0, 0]
    H, W = depth.shape
    di = jnp.clip((depth * (_D - 1)).astype(jnp.int32), 0, _D - 1)

    d0 = db * _DB + jax.lax.broadcasted_iota(jnp.int32, (_DB, 1, 1), 0)
    a = jnp.abs(d0 - di[None, :, :])
    w = jnp.where(a == 0, _K0,
        jnp.where(a == 1, _K1,
        jnp.where(a == 2, _K2,
        jnp.where(a == 3, _K3, 0.0)))).astype(jnp.float32)
    p = w * inten[None, :, :]

    zh = jnp.zeros((_DB, 1, W), jnp.float32)
    zw = jnp.zeros((_DB, H, 1), jnp.float32)
    out_ref[0] = _box3x3(p, zh, zw)

    zh1 = jnp.zeros((1, 1, W), jnp.float32)
    zw1 = jnp.zeros((1, H, 1), jnp.float32)

    @pl.when(db == 0)
    def _():
        g_lo = jnp.where(di == 0, _G1 / 27.0,
               jnp.where(di == 1, _G2 / 27.0, 0.0))
        c = (g_lo * inten)[None]
        out_ref[0, 0:1] -= _box3x3(c, zh1, zw1)[0:1]

    @pl.when(db == nd - 1)
    def _():
        g_hi = jnp.where(di == _D - 1, _G1 / 27.0,
               jnp.where(di == _D - 2, _G2 / 27.0, 0.0))
        c = (g_hi * inten)[None]
        out_ref[0, _DB - 1:_DB] -= _box3x3(c, zh1, zw1)[0:1]


def kernel(depth_map, x_ray):
    B, _, H, W = depth_map.shape
    out = pl.pallas_call(
        _splat_pool_kernel,
        grid=(B, _D // _DB),
        in_specs=[
            pl.BlockSpec((1, 1, H, W), lambda b, d: (b, 0, 0, 0)),
            pl.BlockSpec((1, 1, H, W), lambda b, d: (b, 0, 0, 0)),
        ],
        out_specs=pl.BlockSpec((1, _DB, H, W), lambda b, d: (b, d, 0, 0)),
        out_shape=jax.ShapeDtypeStruct((B, _D, H, W), jnp.float32),
    )(depth_map, x_ray)
    return out[:, None]
